# baseline (speedup 1.0000x reference)
import jax
import jax.numpy as jnp
from jax import lax
from jax.experimental import pallas as pl
from jax.experimental.pallas import tpu as pltpu
from jax.experimental.pallas import tpu_sc as plsc

EMBEDDING_DIM = 128
WINDOW = 128
NBUF = 4
NUM_CORES = 2
NUM_SUBCORES = 16
NUM_WORKERS = NUM_CORES * NUM_SUBCORES


def kernel(sentences_indices, embedding_table):
    batch, hist = sentences_indices.shape
    num_indices = batch * hist
    steps_per_worker = num_indices // (NUM_WORKERS * WINDOW)
    idx2d = sentences_indices.reshape(num_indices // WINDOW, WINDOW).astype(
        jnp.int32
    )

    mesh = plsc.VectorSubcoreMesh(
        core_axis_name="core", subcore_axis_name="subcore"
    )

    @pl.kernel(
        out_type=jax.ShapeDtypeStruct(
            (num_indices // WINDOW, WINDOW, EMBEDDING_DIM), jnp.float32
        ),
        mesh=mesh,
        scratch_types=[
            pltpu.VMEM((steps_per_worker, WINDOW), jnp.int32),
            pltpu.VMEM((NBUF, WINDOW, EMBEDDING_DIM), jnp.float32),
            pltpu.SemaphoreType.DMA((NBUF,)),
            pltpu.SemaphoreType.DMA((NBUF // 2,)),
        ],
    )
    def gather_kernel(table_hbm, idx_hbm, out_hbm, idx_v, bufs, gsem, osem):
        wid = lax.axis_index("subcore") * NUM_CORES + lax.axis_index("core")
        row0 = wid * steps_per_worker

        pltpu.sync_copy(idx_hbm.at[pl.ds(row0, steps_per_worker)], idx_v)

        for b in range(NBUF):
            pltpu.async_copy(table_hbm.at[idx_v.at[b]], bufs.at[b], gsem.at[b])

        def pair_out(p, j):
            for q in range(2):
                pltpu.make_async_copy(
                    table_hbm.at[idx_v.at[j + q]],
                    bufs.at[2 * p + q],
                    gsem.at[2 * p + q],
                ).wait()
            pltpu.async_copy(
                bufs.at[pl.ds(2 * p, 2)],
                out_hbm.at[pl.ds(row0 + j, 2)],
                osem.at[p],
            )

        def pair_out_wait(p, j):
            pltpu.make_async_copy(
                bufs.at[pl.ds(2 * p, 2)],
                out_hbm.at[pl.ds(row0 + j, 2)],
                osem.at[p],
            ).wait()

        @pl.loop(0, steps_per_worker - NBUF, step=NBUF)
        def _(jo):
            for p in range(NBUF // 2):
                j = jo + 2 * p
                pair_out(p, j)
                pair_out_wait(p, j)
                for q in range(2):
                    pltpu.async_copy(
                        table_hbm.at[idx_v.at[j + NBUF + q]],
                        bufs.at[2 * p + q],
                        gsem.at[2 * p + q],
                    )

        jt = steps_per_worker - NBUF
        for p in range(NBUF // 2):
            pair_out(p, jt + 2 * p)
        for p in range(NBUF // 2):
            pair_out_wait(p, jt + 2 * p)

    out = gather_kernel(embedding_table, idx2d)
    return out.reshape(batch, hist, EMBEDDING_DIM)
